# Initial kernel scaffold; baseline (speedup 1.0000x reference)
import jax
import jax.numpy as jnp
from jax.experimental import pallas as pl


def kernel(premise_indices, hypothesis_indices, table_prem, table_hypo, W, b):
    raise NotImplementedError("write your pallas kernel here")


# speedup vs baseline: 2.5880x; 2.5880x over previous
import functools

import jax
import jax.numpy as jnp
from jax import lax
from jax.experimental import pallas as pl
from jax.experimental.pallas import tpu as pltpu
from jax.experimental.pallas import tpu_sc as plsc

VOCAB = 1_000_000
EMB = 64
BATCH = 16384
SEQ = 50
NOUT = 3

NUM_CORES = 2
NUM_SUBCORES = 16
NUM_WORKERS = NUM_CORES * NUM_SUBCORES
ROWS_PER_WORKER = BATCH // NUM_WORKERS
CHUNK = 32
NCHUNKS = ROWS_PER_WORKER // CHUNK
IDX_PER_CHUNK = CHUNK * SEQ


def _sc_pool(idx_p, idx_h, table_p, table_h, dst_tmpl):
  mesh = plsc.VectorSubcoreMesh(core_axis_name="c", subcore_axis_name="s")

  @functools.partial(
      pl.kernel,
      mesh=mesh,
      compiler_params=pltpu.CompilerParams(use_tc_tiling_on_sc=False),
      out_type=(
          jax.ShapeDtypeStruct((BATCH, EMB), jnp.float32),
          jax.ShapeDtypeStruct((BATCH, EMB), jnp.float32),
      ),
      scratch_types=[
          pltpu.VMEM((IDX_PER_CHUNK,), jnp.int32),
          pltpu.VMEM((IDX_PER_CHUNK, EMB), jnp.float32),
          pltpu.VMEM((IDX_PER_CHUNK,), jnp.int32),
          pltpu.VMEM((CHUNK, EMB), jnp.float32),
          pltpu.VMEM_SHARED((NUM_SUBCORES * CHUNK, EMB), jnp.float32),
          pltpu.SemaphoreType.DMA,
      ],
  )
  def pool(idx_p_hbm, idx_h_hbm, tp_hbm, th_hbm, tmpl_hbm,
           out_p_hbm, out_h_hbm,
           idx_v, rows_v, dst_v, zeros_v, acc_s, sem):
    cid = lax.axis_index("c")
    sid = lax.axis_index("s")
    wid = sid * NUM_CORES + cid

    pltpu.sync_copy(tmpl_hbm, dst_v)
    shift = sid * CHUNK
    for i in range(IDX_PER_CHUNK // 16):
      sl = pl.ds(i * 16, 16)
      dst_v[sl] = dst_v[sl] + shift

    for r in range(CHUNK):
      for c16 in range(EMB // 16):
        zeros_v[r, pl.ds(c16 * 16, 16)] = jnp.zeros((16,), jnp.float32)

    def pool_one(idx_hbm, table_hbm, out_hbm):
      def body(c, carry):
        row0 = wid * ROWS_PER_WORKER + c * CHUNK
        pltpu.sync_copy(idx_hbm.at[pl.ds(row0 * SEQ, IDX_PER_CHUNK)], idx_v)
        pltpu.async_copy(table_hbm.at[idx_v], rows_v, sem).wait()
        pltpu.sync_copy(zeros_v, acc_s.at[pl.ds(sid * CHUNK, CHUNK)])
        pltpu.sync_copy(rows_v, acc_s.at[dst_v], add=True)
        pltpu.sync_copy(acc_s.at[pl.ds(sid * CHUNK, CHUNK)],
                        out_hbm.at[pl.ds(row0, CHUNK)])
        return carry

      lax.fori_loop(0, NCHUNKS, body, 0)

    pool_one(idx_p_hbm, tp_hbm, out_p_hbm)
    pool_one(idx_h_hbm, th_hbm, out_h_hbm)

  return pool(idx_p, idx_h, table_p, table_h, dst_tmpl)


_BM = 2048


def _tc_linear(pool_p, pool_h, w, bias):
  def body(p_ref, h_ref, w_ref, b_ref, o_ref):
    wsc = w_ref[...] * jnp.float32(1.0 / SEQ)
    acc = jnp.dot(p_ref[...], wsc[:EMB, :], preferred_element_type=jnp.float32)
    acc += jnp.dot(h_ref[...], wsc[EMB:, :], preferred_element_type=jnp.float32)
    o_ref[...] = acc + b_ref[...]

  return pl.pallas_call(
      body,
      grid=(BATCH // _BM,),
      in_specs=[
          pl.BlockSpec((_BM, EMB), lambda i: (i, 0)),
          pl.BlockSpec((_BM, EMB), lambda i: (i, 0)),
          pl.BlockSpec((2 * EMB, NOUT), lambda i: (0, 0)),
          pl.BlockSpec((1, NOUT), lambda i: (0, 0)),
      ],
      out_specs=pl.BlockSpec((_BM, NOUT), lambda i: (i, 0)),
      out_shape=jax.ShapeDtypeStruct((BATCH, NOUT), jnp.float32),
  )(pool_p, pool_h, w, bias)


def kernel(premise_indices, hypothesis_indices, table_prem, table_hypo, W, b):
  idx_p = premise_indices.astype(jnp.int32).reshape(-1)
  idx_h = hypothesis_indices.astype(jnp.int32).reshape(-1)
  tmpl = jnp.arange(IDX_PER_CHUNK, dtype=jnp.int32) // SEQ
  sums_p, sums_h = _sc_pool(idx_p, idx_h, table_prem, table_hypo, tmpl)
  return _tc_linear(sums_p, sums_h, W, b.reshape(1, NOUT))
